# baseline (speedup 1.0000x reference)
import jax
import jax.numpy as jnp
from jax import lax
from jax.experimental import pallas as pl
from jax.experimental.pallas import tpu as pltpu

_CS = 256


def _attn_body(idx_ref, h_ref, hfull_ref, o_ref, rows, e_sc, sacc, rsem):
    B = idx_ref.shape[0]
    K2 = idx_ref.shape[-1]
    K = K2 // 2
    S = sacc.shape[0]
    C = S // _CS
    b = pl.program_id(0)
    c = pl.program_id(1)

    def row_copy(bb, g):
        return pltpu.make_async_copy(
            hfull_ref.at[bb, idx_ref[bb, g]], rows.at[g], rsem)

    @pl.when((b == 0) & (c == 0))
    def _():
        for g in range(K2):
            row_copy(b, g).start()

    @pl.when(c == 0)
    def _():
        for g in range(K2):
            row_copy(b, g).wait()
        e1 = rows[0, :]
        e2 = rows[K, :]
        for k in range(1, K):
            e1 = e1 + rows[k, :]
            e2 = e2 + rows[K + k, :]
        e_sc[...] = jnp.stack([e1, e2], axis=0)

        @pl.when(b + 1 < B)
        def _():
            for g in range(K2):
                row_copy(b + 1, g).start()

    s = lax.dot_general(
        h_ref[0], e_sc[...], (((1,), (1,)), ((), ())),
        preferred_element_type=jnp.float32,
    )
    sacc[pl.ds(c * _CS, _CS), :] = s

    @pl.when(c == C - 1)
    def _():
        t = sacc[...]
        p = jnp.exp(t - jnp.max(t, axis=0, keepdims=True))
        w = p / jnp.sum(p, axis=0, keepdims=True)
        o_ref[0, 0] = 0.5 * jnp.sum(w, axis=1)


def kernel(input_embed_M, e1_index, e2_index):
    B, S, D = input_embed_M.shape
    K = e1_index.shape[-1]
    eidx = jnp.concatenate(
        [e1_index.astype(jnp.int32), e2_index.astype(jnp.int32)], axis=1
    )
    out = pl.pallas_call(
        _attn_body,
        grid=(B, S // _CS),
        in_specs=[
            pl.BlockSpec(memory_space=pltpu.SMEM),
            pl.BlockSpec((1, _CS, D), lambda b, c: (b, c, 0)),
            pl.BlockSpec(memory_space=pltpu.MemorySpace.HBM),
        ],
        out_specs=pl.BlockSpec((1, 1, S), lambda b, c: (b, 0, 0)),
        out_shape=jax.ShapeDtypeStruct((B, 1, S), jnp.float32),
        scratch_shapes=[
            pltpu.VMEM((2 * K, D), jnp.float32),
            pltpu.VMEM((2, D), jnp.float32),
            pltpu.VMEM((S, 2), jnp.float32),
            pltpu.SemaphoreType.DMA,
        ],
    )(eidx, input_embed_M, input_embed_M)
    return out[:, 0, :]


# speedup vs baseline: 1.6877x; 1.6877x over previous
import jax
import jax.numpy as jnp
from jax import lax
from jax.experimental import pallas as pl
from jax.experimental.pallas import tpu as pltpu

_CS = 1024
_NBUF = 3


def _attn_body(idx_ref, h_ref, o_ref, bufs, rows, sacc, csem, rsem):
    B, S, D = h_ref.shape
    K2 = idx_ref.shape[-1]
    K = K2 // 2
    C = S // _CS
    nchunks = B * C

    def chunk_copy(i):
        b, c = divmod(i, C)
        return pltpu.make_async_copy(
            h_ref.at[b, pl.ds(c * _CS, _CS), :], bufs.at[i % _NBUF],
            csem.at[i % _NBUF])

    def row_copies(b):
        hs = []
        for g in range(K2):
            hs.append(pltpu.make_async_copy(
                h_ref.at[b, idx_ref[b, g]], rows.at[b * K2 + g], rsem))
        return hs

    row_handles = {0: row_copies(0)}
    for h in row_handles[0]:
        h.start()
    handles = []
    for i in range(min(_NBUF, nchunks)):
        handles.append(chunk_copy(i))
        handles[i].start()

    e12 = None
    for i in range(nchunks):
        b, c = divmod(i, C)
        if c == 0:
            for h in row_handles[b]:
                h.wait()
            e1 = rows[b * K2, :]
            e2 = rows[b * K2 + K, :]
            for k in range(1, K):
                e1 = e1 + rows[b * K2 + k, :]
                e2 = e2 + rows[b * K2 + K + k, :]
            e12 = jnp.stack([e1, e2], axis=0)
            if b + 1 < B:
                row_handles[b + 1] = row_copies(b + 1)
                for h in row_handles[b + 1]:
                    h.start()
        handles[i].wait()
        s = lax.dot_general(
            bufs[i % _NBUF], e12, (((1,), (1,)), ((), ())),
            preferred_element_type=jnp.float32,
        )
        sacc[pl.ds(c * _CS, _CS), :] = s
        if i + _NBUF < nchunks:
            handles.append(chunk_copy(i + _NBUF))
            handles[i + _NBUF].start()
        if c == C - 1:
            t = sacc[...]
            p = jnp.exp(t - jnp.max(t, axis=0, keepdims=True))
            w = p / jnp.sum(p, axis=0, keepdims=True)
            o_ref[b, :] = 0.5 * jnp.sum(w, axis=1)


def kernel(input_embed_M, e1_index, e2_index):
    B, S, D = input_embed_M.shape
    K = e1_index.shape[-1]
    eidx = jnp.concatenate(
        [e1_index.astype(jnp.int32), e2_index.astype(jnp.int32)], axis=1
    )
    return pl.pallas_call(
        _attn_body,
        in_specs=[
            pl.BlockSpec(memory_space=pltpu.SMEM),
            pl.BlockSpec(memory_space=pltpu.MemorySpace.HBM),
        ],
        out_specs=pl.BlockSpec(memory_space=pltpu.VMEM),
        out_shape=jax.ShapeDtypeStruct((B, S), jnp.float32),
        scratch_shapes=[
            pltpu.VMEM((_NBUF, _CS, D), jnp.float32),
            pltpu.VMEM((B * 2 * K, D), jnp.float32),
            pltpu.VMEM((S, 2), jnp.float32),
            pltpu.SemaphoreType.DMA((_NBUF,)),
            pltpu.SemaphoreType.DMA,
        ],
    )(eidx, input_embed_M)
